# baseline (speedup 1.0000x reference)
import functools

import jax
import jax.numpy as jnp
from jax import lax
from jax.experimental import pallas as pl
from jax.experimental.pallas import tpu as pltpu
from jax.experimental.pallas import tpu_sc as plsc

NC = 2
NS = 16
NW = NC * NS
C = 128
NBUF = 2
IB = 8
SPLIT = 2


def _sc_degree(col_r, ones128, zeros128, n_acc, K):
    mesh = plsc.VectorSubcoreMesh(core_axis_name="c", subcore_axis_name="s")
    rpt = n_acc // NS

    @functools.partial(
        pl.kernel,
        out_type=jax.ShapeDtypeStruct((NC * n_acc, 128), jnp.float32),
        mesh=mesh,
        scratch_types=[
            pltpu.VMEM((K, C), jnp.int32),
            pltpu.VMEM((C, 128), jnp.float32),
            pltpu.VMEM((C, 128), jnp.float32),
            pltpu.VMEM_SHARED((n_acc, 128), jnp.float32),
            pltpu.SemaphoreType.DMA,
        ],
    )
    def deg_kernel(col_hbm, ones_hbm, zeros_hbm, out_hbm, cidx, ones_v, zeros_v,
                   acc, sem):
        c = lax.axis_index("c")
        s = lax.axis_index("s")
        wid = c * NS + s
        pltpu.sync_copy(col_hbm.at[pl.ds(wid * K, K)], cidx)
        pltpu.sync_copy(ones_hbm, ones_v)
        pltpu.sync_copy(zeros_hbm, zeros_v)
        base = s * rpt
        off = 0
        while off < rpt:
            m = min(C, rpt - off)
            pltpu.sync_copy(zeros_v.at[pl.ds(0, m)], acc.at[pl.ds(base + off, m)])
            off += m
        plsc.subcore_barrier()

        descs = [pltpu.async_copy(ones_v, acc.at[cidx.at[k]], sem, add=True)
                 for k in range(K)]
        for d in descs:
            d.wait()
        plsc.subcore_barrier()
        pltpu.sync_copy(acc.at[pl.ds(base, rpt)],
                        out_hbm.at[pl.ds(c * n_acc + base, rpt)])

    return deg_kernel(col_r, ones128, zeros128)


def _sc_aggregate(y, row_r, col_r, zeros128, n_acc, KA, KB, n, dtype):
    mesh = plsc.VectorSubcoreMesh(core_axis_name="c", subcore_axis_name="s")
    rpt = n_acc // NS
    KC = min(KA, KB)
    KE = max(KA, KB) - KC
    big_core = 0 if KA > KB else 1

    @functools.partial(
        pl.kernel,
        out_type=jax.ShapeDtypeStruct((NC * n_acc, 128), dtype),
        mesh=mesh,
        scratch_types=[
            [pltpu.VMEM((IB, C), jnp.int32)] * 2,
            [pltpu.VMEM((IB, C), jnp.int32)] * 2,
            [pltpu.VMEM((C, 128), dtype)] * NBUF,
            pltpu.VMEM_SHARED((n_acc, 128), dtype),
            [pltpu.SemaphoreType.DMA] * (NBUF * SPLIT),
            [pltpu.SemaphoreType.DMA] * (NBUF * SPLIT),
            [pltpu.SemaphoreType.DMA] * 4,
        ],
    )
    def agg_kernel(y_hbm, row_hbm, col_hbm, z_hbm, out_hbm,
                   ridx, cidx, bufs, acc, gsems, ssems, isems):
        c = lax.axis_index("c")
        s = lax.axis_index("s")
        tbase = jnp.where(c == 0, s * KA, NS * KA + s * KB)
        pltpu.sync_copy(z_hbm, bufs[0])
        base = s * rpt
        off = 0
        while off < rpt:
            m = min(C, rpt - off)
            pltpu.sync_copy(bufs[0].at[pl.ds(0, m)], acc.at[pl.ds(base + off, m)])
            off += m
        plsc.subcore_barrier()

        CS = C // SPLIT

        def run_chunks(base0, nch):
            nblk = nch // IB

            def idx_prefetch(j):
                slot = j % 2
                return (
                    pltpu.async_copy(row_hbm.at[pl.ds(base0 + j * IB, IB)],
                                     ridx[slot], isems[2 * slot]),
                    pltpu.async_copy(col_hbm.at[pl.ds(base0 + j * IB, IB)],
                                     cidx[slot], isems[2 * slot + 1]),
                )

            pltpu.sync_copy(row_hbm.at[pl.ds(base0, IB)], ridx[0])
            pltpu.sync_copy(col_hbm.at[pl.ds(base0, IB)], cidx[0])
            idx_d = [None, None]
            if nblk > 1:
                idx_d[1] = idx_prefetch(1)

            gd = [None] * NBUF
            sd = [None] * NBUF
            state = {"blk": 0}

            def gather(g, b):
                j = g // IB
                if j > state["blk"]:
                    for d in idx_d[j % 2]:
                        d.wait()
                    state["blk"] = j
                gd[b] = [
                    pltpu.async_copy(
                        y_hbm.at[ridx[j % 2].at[g % IB, pl.ds(h * CS, CS)]],
                        bufs[b].at[pl.ds(h * CS, CS)], gsems[b * SPLIT + h])
                    for h in range(SPLIT)
                ]

            for k in range(min(NBUF, nch)):
                gather(k, k)
            for k in range(nch):
                b = k % NBUF
                j = k // IB
                if k % IB == NBUF and j + 1 < nblk and j >= 1:
                    idx_d[(j + 1) % 2] = idx_prefetch(j + 1)
                for d in gd[b]:
                    d.wait()
                sd[b] = [
                    pltpu.async_copy(
                        bufs[b].at[pl.ds(h * CS, CS)],
                        acc.at[cidx[j % 2].at[k % IB, pl.ds(h * CS, CS)]],
                        ssems[b * SPLIT + h], add=True)
                    for h in range(SPLIT)
                ]
                if k + NBUF < nch:
                    for d in sd[b]:
                        d.wait()
                    gather(k + NBUF, b)
            for k in range(max(0, nch - NBUF), nch):
                for d in sd[k % NBUF]:
                    d.wait()

        run_chunks(tbase, KC)
        if KE > 0:
            @pl.when(c == big_core)
            def _extra():
                run_chunks(tbase + KC, KE)

        plsc.subcore_barrier()
        pltpu.sync_copy(acc.at[pl.ds(base, rpt)],
                        out_hbm.at[pl.ds(c * n_acc + base, rpt)])

    return agg_kernel(y, row_r, col_r, zeros128)


def _tc_matmul(x, W):
    def body(x_ref, w_ref, o_ref):
        o_ref[...] = jnp.dot(x_ref[...], w_ref[...],
                             preferred_element_type=jnp.float32)

    return pl.pallas_call(
        body,
        out_shape=jax.ShapeDtypeStruct((x.shape[0], 128), jnp.float32),
    )(x, W)


def _tc_scale(xw, deg_p, n, n_acc):
    def body(xw_ref, d_ref, y_ref):
        d = d_ref[...]
        deg = d[0:n, 0:1] + d[n_acc:n_acc + n, 0:1] + 1.0
        y_ref[...] = xw_ref[...] * lax.rsqrt(deg)

    return pl.pallas_call(
        body,
        out_shape=jax.ShapeDtypeStruct((n, 128), jnp.float32),
    )(xw, deg_p)


def _tc_final(S, y, deg_p, b, n, n_acc):
    def body(s_ref, y_ref, d_ref, b_ref, o_ref):
        d = d_ref[...]
        deg = d[0:n, 0:1] + d[n_acc:n_acc + n, 0:1] + 1.0
        s = s_ref[...].astype(jnp.float32)
        agg = s[0:n, :] + s[n_acc:n_acc + n, :] + y_ref[...]
        o_ref[...] = jnp.maximum(agg * lax.rsqrt(deg) + b_ref[...], 0.0)

    return pl.pallas_call(
        body,
        out_shape=jax.ShapeDtypeStruct((n, 128), jnp.float32),
    )(S, y, deg_p, b.reshape(1, 128))


def kernel(x, edge_index, W, b):
    n = x.shape[0]
    e = edge_index.shape[1]
    K = -(-(-(-e // (NW * C))) // 8) * 8
    e_pad = NW * K * C
    row = edge_index[0]
    col = edge_index[1]
    if e_pad > e:
        row_u = jnp.concatenate([row, jnp.zeros((e_pad - e,), jnp.int32)])
        col_u = jnp.concatenate([col, jnp.full((e_pad - e,), n, jnp.int32)])
    else:
        row_u, col_u = row, col
    col_r = col_u.reshape(NW * K, C)
    n_acc = -(-(n + 1) // (NS * 8)) * (NS * 8)

    KA = max(8, ((2 * K // 4) // 8) * 8)
    KB = 2 * K - KA
    eA = NS * KA * C
    rowA = row_u[:eA].reshape(NS * KA, C)
    colA = col_u[:eA].reshape(NS * KA, C)
    rowB = row_u[eA:].reshape(NS * KB, C)
    colB = col_u[eA:].reshape(NS * KB, C)
    row_r = jnp.concatenate([rowA, rowB])
    col_ra = jnp.concatenate([colA, colB])

    ones128 = jnp.ones((C, 128), jnp.float32)
    zeros128 = jnp.zeros((C, 128), jnp.float32)

    xw = _tc_matmul(x, W)
    deg_p = _sc_degree(col_r, ones128, zeros128, n_acc, K)
    y = _tc_scale(xw, deg_p, n, n_acc)
    S = _sc_aggregate(y, row_r, col_ra, zeros128, n_acc, KA, KB, n,
                      jnp.float32)
    return _tc_final(S, y, deg_p, b, n, n_acc)
